# baseline (speedup 1.0000x reference)
import functools

import jax
import jax.numpy as jnp
from jax import lax
from jax.experimental import pallas as pl
from jax.experimental.pallas import tpu as pltpu
from jax.experimental.pallas import tpu_sc as plsc

N = 10000
E = 320000
D = 128
NC = 2
NS = 16
L = 16
NW = NC * NS

NPD = 10240
NPDT = NPD // NS

NPH = 5120
NPHA = 5248
WROWS = NPH // NS
ZROWS = 80

CHUNK_A = 80
EB = E // NW
CAP = 10240
CAPCH = CAP // CHUNK_A
CAP2 = 2 * CAP
BINBLK = 2000

_sc_mesh = plsc.VectorSubcoreMesh(core_axis_name="c", subcore_axis_name="s")
_sc_params = pltpu.CompilerParams(needs_layout_passes=False)


@functools.partial(
    pl.kernel,
    out_type=[
        jax.ShapeDtypeStruct((NW, 2, 1, CAP2), jnp.int32),
        jax.ShapeDtypeStruct((2, NW, 1, L), jnp.int32),
        jax.ShapeDtypeStruct((NC, NPD), jnp.float32),
    ],
    mesh=_sc_mesh,
    scratch_types=[
        pltpu.VMEM((2, BINBLK), jnp.int32),
        pltpu.VMEM((CAP2,), jnp.int32),
        pltpu.VMEM((CAP2,), jnp.int32),
        pltpu.VMEM((2, 1, L), jnp.int32),
        pltpu.VMEM((NPD,), jnp.float32),
        pltpu.VMEM((NS, NPDT), jnp.float32),
        pltpu.VMEM((NPDT,), jnp.float32),
        pltpu.VMEM_SHARED((NS, NPD), jnp.float32),
    ],
    compiler_params=_sc_params,
)
def _bin_sc(edges_hbm, out_hbm, cnt_hbm, deg_hbm,
            in_v, f0_v, f1_v, cb_v, hist_v, red_v, ob_v, sh_v):
    c = lax.axis_index("c")
    s = lax.axis_index("s")
    wid = s * NC + c

    izero = jnp.zeros((L,), jnp.int32)
    ienn = jnp.full((L,), N, jnp.int32)

    @pl.loop(0, CAP2, step=2 * CHUNK_A)
    def _(i):
        for j in range(0, CHUNK_A, L):
            f0_v[pl.ds(i + j, L)] = izero
            f1_v[pl.ds(i + j, L)] = izero
            f0_v[pl.ds(i + CHUNK_A + j, L)] = ienn
            f1_v[pl.ds(i + CHUNK_A + j, L)] = ienn

    @pl.loop(0, NPD, step=L)
    def _(i):
        hist_v[pl.ds(i, L)] = jnp.zeros((L,), jnp.float32)

    ones = jnp.ones((L,), jnp.float32)

    def grp(g, carry):
        p0, p1 = carry
        sv = in_v[0, pl.ds(g * L, L)]
        dv = in_v[1, pl.ds(g * L, L)]
        plsc.addupdate_scatter(hist_v, [dv], ones)
        m0 = dv < NPH
        m0i = m0.astype(jnp.int32)
        pf0 = plsc.cumsum(m0i)
        n0 = jnp.sum(m0i)
        idx0 = p0 + pf0 - 1
        fl0 = (idx0 // CHUNK_A) * (2 * CHUNK_A) + (idx0 % CHUNK_A)
        plsc.store_scatter(f0_v, [fl0], sv, mask=m0)
        plsc.store_scatter(f0_v, [fl0 + CHUNK_A], dv, mask=m0)
        m1 = jnp.logical_not(m0)
        m1i = m1.astype(jnp.int32)
        pf1 = plsc.cumsum(m1i)
        idx1 = p1 + pf1 - 1
        fl1 = (idx1 // CHUNK_A) * (2 * CHUNK_A) + (idx1 % CHUNK_A)
        plsc.store_scatter(f1_v, [fl1], sv, mask=m1)
        plsc.store_scatter(f1_v, [fl1 + CHUNK_A], dv, mask=m1)
        return p0 + n0, p1 + (L - n0)

    def block(b, carry):
        pltpu.sync_copy(edges_hbm.at[wid, b], in_v)
        return lax.fori_loop(0, BINBLK // L, grp, carry)

    p0, p1 = lax.fori_loop(0, EB // BINBLK, block,
                           (jnp.int32(0), jnp.int32(0)))

    def padded_chunks(p):
        ch = (p + (CHUNK_A - 1)) // CHUNK_A
        return jnp.maximum(((ch + 7) // 8) * 8, 16)

    cb_v[0, 0, pl.ds(0, L)] = izero + padded_chunks(p0)
    cb_v[1, 0, pl.ds(0, L)] = izero + padded_chunks(p1)

    pltpu.sync_copy(f0_v, out_hbm.at[wid, 0, 0])
    pltpu.sync_copy(f1_v, out_hbm.at[wid, 1, 0])
    pltpu.sync_copy(cb_v.at[0], cnt_hbm.at[0, wid])
    pltpu.sync_copy(cb_v.at[1], cnt_hbm.at[1, wid])

    pltpu.sync_copy(hist_v, sh_v.at[s])
    plsc.subcore_barrier()

    col0 = s * NPDT
    for r in range(NS):
        pltpu.sync_copy(sh_v.at[r, pl.ds(col0, NPDT)], red_v.at[r])

    @pl.loop(0, NPDT, step=L)
    def _(j):
        acc = red_v[0, pl.ds(j, L)]
        for r in range(1, NS):
            acc = acc + red_v[r, pl.ds(j, L)]
        ob_v[pl.ds(j, L)] = acc

    pltpu.sync_copy(ob_v, deg_hbm.at[c, pl.ds(col0, NPDT)])


NBR = 4
NBI = 8
UNROLL = 8


@functools.partial(
    pl.kernel,
    out_type=jax.ShapeDtypeStruct((NC, NPH, D), jnp.float32),
    mesh=_sc_mesh,
    scratch_types=[
        pltpu.VMEM((NBI, 2, CHUNK_A), jnp.int32),
        pltpu.VMEM((NBR, CHUNK_A, D), jnp.float32),
        pltpu.VMEM((1, L), jnp.int32),
        pltpu.SemaphoreType.DMA((NBI,)),
        pltpu.SemaphoreType.DMA((NBR,)),
        pltpu.SemaphoreType.DMA((NBR,)),
        pltpu.VMEM_SHARED((NPHA, D), jnp.float32),
    ],
    compiler_params=_sc_params,
)
def _agg_sc(binned_hbm, cnts_hbm, data_hbm, out_hbm,
            idx_v, rows_v, cnt_v, isem, gsem, ssem, acc_sh):
    c = lax.axis_index("c")
    s = lax.axis_index("s")

    @pl.loop(0, ZROWS)
    def _(r):
        @pl.loop(0, D, step=L)
        def _(j):
            rows_v[0, r, pl.ds(j, L)] = jnp.zeros((L,), jnp.float32)

    row0 = s * WROWS
    for b in range(WROWS // ZROWS):
        pltpu.sync_copy(rows_v.at[0, pl.ds(0, ZROWS)],
                        acc_sh.at[pl.ds(row0 + b * ZROWS, ZROWS)])
    plsc.subcore_barrier()

    base = c * NPH
    trash = jnp.full((L,), NPH, jnp.int32) + lax.iota(jnp.int32, L)

    def run_sublist(bt):
        def start_idx(i, bi):
            pltpu.async_copy(binned_hbm.at[bt, c, i], idx_v.at[bi],
                             isem.at[bi])

        def wait_idx(i, bi):
            pltpu.make_async_copy(binned_hbm.at[bt, c, i], idx_v.at[bi],
                                  isem.at[bi]).wait()

        def translate(bi):
            for j in range(0, CHUNK_A, L):
                d = idx_v[bi, 1, pl.ds(j, L)]
                local = d - base
                inb = (local >= 0) & (local < NPH)
                idx_v[bi, 1, pl.ds(j, L)] = jnp.where(inb, local, trash)

        def start_g(br, bi):
            pltpu.async_copy(data_hbm.at[idx_v.at[bi, 0]], rows_v.at[br],
                             gsem.at[br])

        def wait_g(br, bi):
            pltpu.make_async_copy(data_hbm.at[idx_v.at[bi, 0]], rows_v.at[br],
                                  gsem.at[br]).wait()

        def start_s(br, bi):
            pltpu.async_copy(rows_v.at[br], acc_sh.at[idx_v.at[bi, 1]],
                             ssem.at[br], add=True)

        def wait_s(br, bi):
            pltpu.make_async_copy(rows_v.at[br], acc_sh.at[idx_v.at[bi, 1]],
                                  ssem.at[br]).wait()

        def slot(j, r, first=True, idx6=True, head=False):
            if first:
                b2r, b2i = (r + 2) % NBR, (r + 2) % NBI
                wait_idx(j + 2, b2i)
                translate(b2i)
                if not head:
                    wait_s(b2r, b2i)
                start_g(b2r, b2i)
            wait_g(r % NBR, r % NBI)
            start_s(r % NBR, r % NBI)
            if idx6:
                start_idx(j + 6, (r + 6) % NBI)

        pltpu.sync_copy(cnts_hbm.at[c, bt], cnt_v)
        nch = cnt_v[0, pl.ds(0, L)][0]

        for i in range(6):
            start_idx(i, i)
        wait_idx(0, 0)
        translate(0)
        wait_idx(1, 1)
        translate(1)
        start_g(0, 0)
        start_g(1, 1)

        for j in range(UNROLL):
            slot(j, j, head=(j < NBR - 2))

        @pl.loop(1, nch // UNROLL - 1)
        def _(k):
            for r in range(UNROLL):
                slot(k * UNROLL + r, r)

        for r in range(UNROLL):
            slot(nch - UNROLL + r, r, first=(r < UNROLL - 2), idx6=(r < 2))

        for r in range(NBR):
            wait_s(r, (NBR + r) % NBI)

    run_sublist(s * NC)
    run_sublist(s * NC + 1)

    plsc.subcore_barrier()
    pltpu.sync_copy(acc_sh.at[pl.ds(row0, WROWS)], out_hbm.at[c, pl.ds(row0, WROWS)])


BN = 400
GRID = N // BN


def _pre_body(deg_ref, x_ref, xs_ref, dis_ref):
    deg = deg_ref[0] + deg_ref[1] + 1.0
    dis = lax.rsqrt(deg)
    dis_ref[...] = dis
    xs_ref[...] = x_ref[...] * dis


_pre_tc = pl.pallas_call(
    _pre_body,
    grid=(GRID,),
    in_specs=[
        pl.BlockSpec((2, BN, 1), lambda i: (0, i, 0)),
        pl.BlockSpec((BN, D), lambda i: (i, 0)),
    ],
    out_specs=[
        pl.BlockSpec((BN, D), lambda i: (i, 0)),
        pl.BlockSpec((BN, 1), lambda i: (i, 0)),
    ],
    out_shape=[
        jax.ShapeDtypeStruct((N, D), jnp.float32),
        jax.ShapeDtypeStruct((N, 1), jnp.float32),
    ],
)


def _mm_body(p_ref, xs_ref, dis_ref, w1_ref, b1_ref, w2_ref, t_ref):
    dis = dis_ref[...]
    z = (p_ref[...] + xs_ref[...]) * dis
    h = jnp.dot(z, w1_ref[...], preferred_element_type=jnp.float32) + b1_ref[...]
    h = jnp.maximum(h, 0.0)
    t_ref[...] = jnp.dot(h, w2_ref[...], preferred_element_type=jnp.float32) * dis


_mm_tc = pl.pallas_call(
    _mm_body,
    grid=(GRID,),
    in_specs=[
        pl.BlockSpec((BN, D), lambda i: (i, 0)),
        pl.BlockSpec((BN, D), lambda i: (i, 0)),
        pl.BlockSpec((BN, 1), lambda i: (i, 0)),
        pl.BlockSpec((D, 2 * D), lambda i: (0, 0)),
        pl.BlockSpec((1, 2 * D), lambda i: (0, 0)),
        pl.BlockSpec((2 * D, D), lambda i: (0, 0)),
    ],
    out_specs=pl.BlockSpec((BN, D), lambda i: (i, 0)),
    out_shape=jax.ShapeDtypeStruct((N, D), jnp.float32),
)


def _out_body(p_ref, t_ref, dis_ref, b2_ref, o_ref):
    o_ref[...] = (p_ref[...] + t_ref[...]) * dis_ref[...] + b2_ref[...]


_out_tc = pl.pallas_call(
    _out_body,
    grid=(GRID,),
    in_specs=[
        pl.BlockSpec((BN, D), lambda i: (i, 0)),
        pl.BlockSpec((BN, D), lambda i: (i, 0)),
        pl.BlockSpec((BN, 1), lambda i: (i, 0)),
        pl.BlockSpec((1, D), lambda i: (0, 0)),
    ],
    out_specs=pl.BlockSpec((BN, D), lambda i: (i, 0)),
    out_shape=jax.ShapeDtypeStruct((N, D), jnp.float32),
)


def kernel(x, edge_index, conv1_weight, conv1_bias, conv2_weight, conv2_bias):
    nblk = EB // BINBLK
    edges2 = jnp.stack([edge_index[0].reshape(NW, nblk, BINBLK),
                        edge_index[1].reshape(NW, nblk, BINBLK)], axis=2)

    binned, cnts, deg_parts = _bin_sc(edges2)
    binned = binned.reshape(NW, 2, CAPCH, 2, CHUNK_A)
    deg2 = deg_parts[:, :N].reshape(2, N, 1)
    xs, dis = _pre_tc(deg2, x)
    p1 = _agg_sc(binned, cnts, xs).reshape(NC * NPH, D)
    t = _mm_tc(p1, xs, dis, conv1_weight,
               conv1_bias.reshape(1, 2 * D), conv2_weight)
    p2 = _agg_sc(binned, cnts, t).reshape(NC * NPH, D)
    out = _out_tc(p2, t, dis, conv2_bias.reshape(1, D))
    return out


# speedup vs baseline: 1.0001x; 1.0001x over previous
import functools

import jax
import jax.numpy as jnp
from jax import lax
from jax.experimental import pallas as pl
from jax.experimental.pallas import tpu as pltpu
from jax.experimental.pallas import tpu_sc as plsc

N = 10000
E = 320000
D = 128
NC = 2
NS = 16
L = 16
NW = NC * NS

NPD = 10240
NPDT = NPD // NS

NPH = 5120
NPHA = 5248
WROWS = NPH // NS
ZROWS = 80

CHUNK_A = 80
EB = E // NW
CAP = 10240
CAPCH = CAP // CHUNK_A
CAP2 = 2 * CAP
BINBLK = 2000

_sc_mesh = plsc.VectorSubcoreMesh(core_axis_name="c", subcore_axis_name="s")
_sc_params = pltpu.CompilerParams(needs_layout_passes=False)


@functools.partial(
    pl.kernel,
    out_type=[
        jax.ShapeDtypeStruct((NW, 2, 1, CAP2), jnp.int32),
        jax.ShapeDtypeStruct((2, NW, 1, L), jnp.int32),
        jax.ShapeDtypeStruct((NC, NPD), jnp.float32),
    ],
    mesh=_sc_mesh,
    scratch_types=[
        pltpu.VMEM((2, BINBLK), jnp.int32),
        pltpu.VMEM((CAP2,), jnp.int32),
        pltpu.VMEM((CAP2,), jnp.int32),
        pltpu.VMEM((2, 1, L), jnp.int32),
        pltpu.VMEM((NPD,), jnp.float32),
        pltpu.VMEM((NS, NPDT), jnp.float32),
        pltpu.VMEM((NPDT,), jnp.float32),
        pltpu.VMEM_SHARED((NS, NPD), jnp.float32),
    ],
    compiler_params=_sc_params,
)
def _bin_sc(edges_hbm, out_hbm, cnt_hbm, deg_hbm,
            in_v, f0_v, f1_v, cb_v, hist_v, red_v, ob_v, sh_v):
    c = lax.axis_index("c")
    s = lax.axis_index("s")
    wid = s * NC + c

    izero = jnp.zeros((L,), jnp.int32)
    ienn = jnp.full((L,), N, jnp.int32)

    @pl.loop(0, CAP2, step=2 * CHUNK_A)
    def _(i):
        for j in range(0, CHUNK_A, L):
            f0_v[pl.ds(i + j, L)] = izero
            f1_v[pl.ds(i + j, L)] = izero
            f0_v[pl.ds(i + CHUNK_A + j, L)] = ienn
            f1_v[pl.ds(i + CHUNK_A + j, L)] = ienn

    @pl.loop(0, NPD, step=L)
    def _(i):
        hist_v[pl.ds(i, L)] = jnp.zeros((L,), jnp.float32)

    ones = jnp.ones((L,), jnp.float32)

    def grp(g, carry):
        p0, p1 = carry
        sv = in_v[0, pl.ds(g * L, L)]
        dv = in_v[1, pl.ds(g * L, L)]
        plsc.addupdate_scatter(hist_v, [dv], ones)
        m0 = dv < NPH
        m0i = m0.astype(jnp.int32)
        pf0 = plsc.cumsum(m0i)
        n0 = jnp.sum(m0i)
        idx0 = p0 + pf0 - 1
        fl0 = (idx0 // CHUNK_A) * (2 * CHUNK_A) + (idx0 % CHUNK_A)
        plsc.store_scatter(f0_v, [fl0], sv, mask=m0)
        plsc.store_scatter(f0_v, [fl0 + CHUNK_A], dv, mask=m0)
        m1 = jnp.logical_not(m0)
        m1i = m1.astype(jnp.int32)
        pf1 = plsc.cumsum(m1i)
        idx1 = p1 + pf1 - 1
        fl1 = (idx1 // CHUNK_A) * (2 * CHUNK_A) + (idx1 % CHUNK_A)
        plsc.store_scatter(f1_v, [fl1], sv, mask=m1)
        plsc.store_scatter(f1_v, [fl1 + CHUNK_A], dv, mask=m1)
        return p0 + n0, p1 + (L - n0)

    def block(b, carry):
        pltpu.sync_copy(edges_hbm.at[wid, b], in_v)
        return lax.fori_loop(0, BINBLK // L, grp, carry)

    p0, p1 = lax.fori_loop(0, EB // BINBLK, block,
                           (jnp.int32(0), jnp.int32(0)))

    def padded_chunks(p):
        ch = (p + (CHUNK_A - 1)) // CHUNK_A
        return jnp.maximum(((ch + 7) // 8) * 8, 16)

    cb_v[0, 0, pl.ds(0, L)] = izero + padded_chunks(p0)
    cb_v[1, 0, pl.ds(0, L)] = izero + padded_chunks(p1)

    pltpu.sync_copy(f0_v, out_hbm.at[wid, 0, 0])
    pltpu.sync_copy(f1_v, out_hbm.at[wid, 1, 0])
    pltpu.sync_copy(cb_v.at[0], cnt_hbm.at[0, wid])
    pltpu.sync_copy(cb_v.at[1], cnt_hbm.at[1, wid])

    pltpu.sync_copy(hist_v, sh_v.at[s])
    plsc.subcore_barrier()

    col0 = s * NPDT
    for r in range(NS):
        pltpu.sync_copy(sh_v.at[r, pl.ds(col0, NPDT)], red_v.at[r])

    @pl.loop(0, NPDT, step=L)
    def _(j):
        acc = red_v[0, pl.ds(j, L)]
        for r in range(1, NS):
            acc = acc + red_v[r, pl.ds(j, L)]
        ob_v[pl.ds(j, L)] = acc

    pltpu.sync_copy(ob_v, deg_hbm.at[c, pl.ds(col0, NPDT)])


NBR = 4
NBI = 8
UNROLL = 8


@functools.partial(
    pl.kernel,
    out_type=jax.ShapeDtypeStruct((NC, NPH, D), jnp.float32),
    mesh=_sc_mesh,
    scratch_types=[
        pltpu.VMEM((NBI, 2, CHUNK_A), jnp.int32),
        pltpu.VMEM((NBR, CHUNK_A, D), jnp.float32),
        pltpu.VMEM((1, L), jnp.int32),
        pltpu.SemaphoreType.DMA((NBI,)),
        pltpu.SemaphoreType.DMA((NBR,)),
        pltpu.SemaphoreType.DMA((NBR,)),
        pltpu.VMEM_SHARED((NPHA, D), jnp.float32),
    ],
    compiler_params=_sc_params,
)
def _agg_sc(binned_hbm, cnts_hbm, data_hbm, out_hbm,
            idx_v, rows_v, cnt_v, isem, gsem, ssem, acc_sh):
    c = lax.axis_index("c")
    s = lax.axis_index("s")

    @pl.loop(0, ZROWS)
    def _(r):
        @pl.loop(0, D, step=L)
        def _(j):
            rows_v[0, r, pl.ds(j, L)] = jnp.zeros((L,), jnp.float32)

    row0 = s * WROWS
    for b in range(WROWS // ZROWS):
        pltpu.sync_copy(rows_v.at[0, pl.ds(0, ZROWS)],
                        acc_sh.at[pl.ds(row0 + b * ZROWS, ZROWS)])
    plsc.subcore_barrier()

    base = c * NPH
    trash = jnp.full((L,), NPH, jnp.int32) + lax.iota(jnp.int32, L)

    def run_sublist(bt):
        def start_idx(i, bi):
            pltpu.async_copy(binned_hbm.at[bt, c, i], idx_v.at[bi],
                             isem.at[bi])

        def wait_idx(i, bi):
            pltpu.make_async_copy(binned_hbm.at[bt, c, i], idx_v.at[bi],
                                  isem.at[bi]).wait()

        def translate(bi):
            for j in range(0, CHUNK_A, L):
                d = idx_v[bi, 1, pl.ds(j, L)]
                local = d - base
                inb = (local >= 0) & (local < NPH)
                idx_v[bi, 1, pl.ds(j, L)] = jnp.where(inb, local, trash)

        def start_g(br, bi):
            pltpu.async_copy(data_hbm.at[idx_v.at[bi, 0]], rows_v.at[br],
                             gsem.at[br])

        def wait_g(br, bi):
            pltpu.make_async_copy(data_hbm.at[idx_v.at[bi, 0]], rows_v.at[br],
                                  gsem.at[br]).wait()

        def start_s(br, bi):
            pltpu.async_copy(rows_v.at[br], acc_sh.at[idx_v.at[bi, 1]],
                             ssem.at[br], add=True)

        def wait_s(br, bi):
            pltpu.make_async_copy(rows_v.at[br], acc_sh.at[idx_v.at[bi, 1]],
                                  ssem.at[br]).wait()

        def slot(j, r, first=True, idx6=True, head=False):
            if first:
                b2r, b2i = (r + 2) % NBR, (r + 2) % NBI
                wait_idx(j + 2, b2i)
                translate(b2i)
                if not head:
                    wait_s(b2r, b2i)
                start_g(b2r, b2i)
            wait_g(r % NBR, r % NBI)
            start_s(r % NBR, r % NBI)
            if idx6:
                start_idx(j + 6, (r + 6) % NBI)

        pltpu.sync_copy(cnts_hbm.at[c, bt], cnt_v)
        nch = cnt_v[0, pl.ds(0, L)][0]

        for i in range(6):
            start_idx(i, i)
        wait_idx(0, 0)
        translate(0)
        wait_idx(1, 1)
        translate(1)
        start_g(0, 0)
        start_g(1, 1)

        for j in range(UNROLL):
            slot(j, j, head=(j < NBR - 2))

        @pl.loop(1, CAPCH // UNROLL)
        def _(k):
            for r in range(UNROLL):
                j = k * UNROLL + r
                b2r, b2i = (r + 2) % NBR, (r + 2) % NBI

                @pl.when(j + 2 < nch)
                def _():
                    wait_idx(j + 2, b2i)
                    translate(b2i)
                    wait_s(b2r, b2i)
                    start_g(b2r, b2i)

                @pl.when(j < nch)
                def _():
                    wait_g(r % NBR, r % NBI)
                    start_s(r % NBR, r % NBI)

                @pl.when(j + 6 < nch)
                def _():
                    start_idx(j + 6, (r + 6) % NBI)

        for r in range(NBR):
            wait_s(r, (NBR + r) % NBI)

    run_sublist(s * NC)
    run_sublist(s * NC + 1)

    plsc.subcore_barrier()
    pltpu.sync_copy(acc_sh.at[pl.ds(row0, WROWS)], out_hbm.at[c, pl.ds(row0, WROWS)])


BN = 400
GRID = N // BN


def _pre_body(deg_ref, x_ref, xs_ref, dis_ref):
    deg = deg_ref[0] + deg_ref[1] + 1.0
    dis = lax.rsqrt(deg)
    dis_ref[...] = dis
    xs_ref[...] = x_ref[...] * dis


_pre_tc = pl.pallas_call(
    _pre_body,
    grid=(GRID,),
    in_specs=[
        pl.BlockSpec((2, BN, 1), lambda i: (0, i, 0)),
        pl.BlockSpec((BN, D), lambda i: (i, 0)),
    ],
    out_specs=[
        pl.BlockSpec((BN, D), lambda i: (i, 0)),
        pl.BlockSpec((BN, 1), lambda i: (i, 0)),
    ],
    out_shape=[
        jax.ShapeDtypeStruct((N, D), jnp.float32),
        jax.ShapeDtypeStruct((N, 1), jnp.float32),
    ],
)


def _mm_body(p_ref, xs_ref, dis_ref, w1_ref, b1_ref, w2_ref, t_ref):
    dis = dis_ref[...]
    z = (p_ref[...] + xs_ref[...]) * dis
    h = jnp.dot(z, w1_ref[...], preferred_element_type=jnp.float32) + b1_ref[...]
    h = jnp.maximum(h, 0.0)
    t_ref[...] = jnp.dot(h, w2_ref[...], preferred_element_type=jnp.float32) * dis


_mm_tc = pl.pallas_call(
    _mm_body,
    grid=(GRID,),
    in_specs=[
        pl.BlockSpec((BN, D), lambda i: (i, 0)),
        pl.BlockSpec((BN, D), lambda i: (i, 0)),
        pl.BlockSpec((BN, 1), lambda i: (i, 0)),
        pl.BlockSpec((D, 2 * D), lambda i: (0, 0)),
        pl.BlockSpec((1, 2 * D), lambda i: (0, 0)),
        pl.BlockSpec((2 * D, D), lambda i: (0, 0)),
    ],
    out_specs=pl.BlockSpec((BN, D), lambda i: (i, 0)),
    out_shape=jax.ShapeDtypeStruct((N, D), jnp.float32),
)


def _out_body(p_ref, t_ref, dis_ref, b2_ref, o_ref):
    o_ref[...] = (p_ref[...] + t_ref[...]) * dis_ref[...] + b2_ref[...]


_out_tc = pl.pallas_call(
    _out_body,
    grid=(GRID,),
    in_specs=[
        pl.BlockSpec((BN, D), lambda i: (i, 0)),
        pl.BlockSpec((BN, D), lambda i: (i, 0)),
        pl.BlockSpec((BN, 1), lambda i: (i, 0)),
        pl.BlockSpec((1, D), lambda i: (0, 0)),
    ],
    out_specs=pl.BlockSpec((BN, D), lambda i: (i, 0)),
    out_shape=jax.ShapeDtypeStruct((N, D), jnp.float32),
)


def kernel(x, edge_index, conv1_weight, conv1_bias, conv2_weight, conv2_bias):
    nblk = EB // BINBLK
    edges2 = jnp.stack([edge_index[0].reshape(NW, nblk, BINBLK),
                        edge_index[1].reshape(NW, nblk, BINBLK)], axis=2)

    binned, cnts, deg_parts = _bin_sc(edges2)
    binned = binned.reshape(NW, 2, CAPCH, 2, CHUNK_A)
    deg2 = deg_parts[:, :N].reshape(2, N, 1)
    xs, dis = _pre_tc(deg2, x)
    p1 = _agg_sc(binned, cnts, xs).reshape(NC * NPH, D)
    t = _mm_tc(p1, xs, dis, conv1_weight,
               conv1_bias.reshape(1, 2 * D), conv2_weight)
    p2 = _agg_sc(binned, cnts, t).reshape(NC * NPH, D)
    out = _out_tc(p2, t, dis, conv2_bias.reshape(1, D))
    return out
